# baseline (speedup 1.0000x reference)
import functools

import jax
import jax.numpy as jnp
from jax import lax
from jax.experimental import pallas as pl
from jax.experimental.pallas import tpu as pltpu
from jax.experimental.pallas import tpu_sc as plsc

NC = 2
NS = 16
NW = NC * NS

V = 256
D = 64

B = 1024 * 200 * 4
BPW = B // NW
CH = 512
NCHUNK = BPW // CH

_mesh = plsc.VectorSubcoreMesh(core_axis_name="c", subcore_axis_name="s")


@functools.partial(
    pl.kernel,
    mesh=_mesh,
    compiler_params=pltpu.CompilerParams(needs_layout_passes=False),
    out_type=jax.ShapeDtypeStruct((B * D,), jnp.float32),
    scratch_types=[
        pltpu.VMEM((V * D,), jnp.float32),
        pltpu.VMEM((BPW,), jnp.int32),
        pltpu.VMEM((CH * D,), jnp.float32),
        pltpu.VMEM((CH * D,), jnp.float32),
        pltpu.SemaphoreType.DMA,
        pltpu.SemaphoreType.DMA,
    ],
)
def _gather_kernel(idx_hbm, w_hbm, out_hbm, table_v, idx_v, rows0, rows1,
                   sem0, sem1):
    wid = lax.axis_index("s") * NC + lax.axis_index("c")
    base = wid * BPW

    pltpu.sync_copy(w_hbm, table_v)
    pltpu.sync_copy(idx_hbm.at[pl.ds(base, BPW)], idx_v)

    bufs = ((rows0, sem0), (rows1, sem1))

    def fill_and_send(c, rows_b, sem_b):
        out_sl = out_hbm.at[pl.ds((base + c * CH) * D, CH * D)]

        @pl.when(c >= 2)
        def _():
            pltpu.make_async_copy(rows_b, out_sl, sem_b).wait()

        out_lane = lax.iota(jnp.int32, 16) * D

        def row_fn(g, _):
            iv = idx_v[pl.ds(c * CH + g * 16, 16)] * D
            ob = out_lane + g * (16 * D)
            for k in range(D):
                vals = plsc.load_gather(table_v, [iv + k])
                plsc.store_scatter(rows_b, [ob + k], vals)
            return 0

        lax.fori_loop(0, CH // 16, row_fn, 0)
        pltpu.async_copy(rows_b, out_sl, sem_b)

    def round_fn(i, _):
        c0 = i * 2
        for b in range(2):
            fill_and_send(c0 + b, *bufs[b])
        return 0

    lax.fori_loop(0, NCHUNK // 2, round_fn, 0)

    for b in range(2):
        c_last = NCHUNK - 2 + b
        out_sl = out_hbm.at[pl.ds((base + c_last * CH) * D, CH * D)]
        pltpu.make_async_copy(bufs[b][0], out_sl, bufs[b][1]).wait()


def kernel(byte_indices, W):
    batch, seq, four = byte_indices.shape
    idx_flat = byte_indices.reshape(-1).astype(jnp.int32)
    w_flat = W.reshape(-1).astype(jnp.float32)
    out_flat = _gather_kernel(idx_flat, w_flat)
    return out_flat.reshape(batch, seq, four * W.shape[1])


# speedup vs baseline: 4.7078x; 4.7078x over previous
import functools

import jax
import jax.numpy as jnp
from jax import lax
from jax.experimental import pallas as pl
from jax.experimental.pallas import tpu as pltpu
from jax.experimental.pallas import tpu_sc as plsc

NC = 2
NS = 16
NW = NC * NS

V = 256
D = 64

B = 1024 * 200 * 4
BPW = B // NW
CH = 512
NCHUNK = BPW // CH

_mesh = plsc.VectorSubcoreMesh(core_axis_name="c", subcore_axis_name="s")


@functools.partial(
    pl.kernel,
    mesh=_mesh,
    out_type=jax.ShapeDtypeStruct((B * D,), jnp.float32),
    scratch_types=[
        pltpu.VMEM((V * D,), jnp.float32),
        pltpu.VMEM((BPW,), jnp.int32),
        pltpu.VMEM((CH * D,), jnp.float32),
        pltpu.VMEM((CH * D,), jnp.float32),
        pltpu.SemaphoreType.DMA,
        pltpu.SemaphoreType.DMA,
    ],
)
def _gather_kernel(idx_hbm, w_hbm, out_hbm, table_v, idx_v, rows0, rows1,
                   sem0, sem1):
    wid = lax.axis_index("s") * NC + lax.axis_index("c")
    base = wid * BPW

    pltpu.sync_copy(w_hbm, table_v)
    pltpu.sync_copy(idx_hbm.at[pl.ds(base, BPW)], idx_v)

    bufs = ((rows0, sem0), (rows1, sem1))

    def fill_and_send(c, rows_b, sem_b):
        out_sl = out_hbm.at[pl.ds((base + c * CH) * D, CH * D)]

        @pl.when(c >= 2)
        def _():
            pltpu.make_async_copy(rows_b, out_sl, sem_b).wait()

        @plsc.parallel_loop(0, CH // 16, unroll=2)
        def row_fn(g):
            iv = idx_v[pl.ds(c * CH + g * 16, 16)] * D
            rb = g * (16 * D)
            for j in range(16):
                off = iv[j]
                for k in range(D // 16):
                    rows_b[pl.ds(rb + j * D + k * 16, 16)] = (
                        table_v[pl.ds(off + k * 16, 16)])

        pltpu.async_copy(rows_b, out_sl, sem_b)

    def round_fn(i, _):
        c0 = i * 2
        for b in range(2):
            fill_and_send(c0 + b, *bufs[b])
        return 0

    lax.fori_loop(0, NCHUNK // 2, round_fn, 0)

    for b in range(2):
        c_last = NCHUNK - 2 + b
        out_sl = out_hbm.at[pl.ds((base + c_last * CH) * D, CH * D)]
        pltpu.make_async_copy(bufs[b][0], out_sl, bufs[b][1]).wait()


def kernel(byte_indices, W):
    batch, seq, four = byte_indices.shape
    idx_flat = byte_indices.reshape(-1).astype(jnp.int32)
    w_flat = W.reshape(-1).astype(jnp.float32)
    out_flat = _gather_kernel(idx_flat, w_flat)
    return out_flat.reshape(batch, seq, four * W.shape[1])


# speedup vs baseline: 5.2819x; 1.1219x over previous
import functools

import jax
import jax.numpy as jnp
from jax import lax
from jax.experimental import pallas as pl
from jax.experimental.pallas import tpu as pltpu
from jax.experimental.pallas import tpu_sc as plsc

NC = 2
NS = 16
NW = NC * NS

V = 256
D = 64
BATCH = 1024
SEQ = 200
K = 4
NB = BATCH // NW
RPB = SEQ * K
CHS = 40
NCH = SEQ // CHS
CGRP = CHS * K // 16
TCH = NB * NCH

_mesh = plsc.VectorSubcoreMesh(core_axis_name="c", subcore_axis_name="s")


@functools.partial(
    pl.kernel,
    mesh=_mesh,
    out_type=jax.ShapeDtypeStruct((BATCH, SEQ, K * D), jnp.float32),
    scratch_types=[
        pltpu.VMEM((V, D), jnp.float32),
        pltpu.VMEM((RPB,), jnp.int32),
        pltpu.VMEM((CHS, K * D), jnp.float32),
        pltpu.VMEM((CHS, K * D), jnp.float32),
        pltpu.SemaphoreType.DMA,
        pltpu.SemaphoreType.DMA,
    ],
)
def _gather_kernel(idx_hbm, w_hbm, out_hbm, table_v, idx_b, rows0, rows1,
                   sem0, sem1):
    wid = lax.axis_index("s") * NC + lax.axis_index("c")
    base_b = wid * NB

    pltpu.sync_copy(w_hbm, table_v)

    bufs = ((rows0, sem0), (rows1, sem1))

    def do_chunk(t, rows_b, sem_b):
        bb = base_b + t // NCH
        h = t % NCH
        out_sl = out_hbm.at[bb, pl.ds(h * CHS, CHS), :]

        @pl.when(h == 0)
        def _():
            pltpu.sync_copy(idx_hbm.at[pl.ds(bb * RPB, RPB)], idx_b)

        @pl.when(t >= 2)
        def _():
            pltpu.make_async_copy(rows_b, out_sl, sem_b).wait()

        @plsc.parallel_loop(0, CGRP, unroll=2)
        def grp_fn(g):
            iv = idx_b[pl.ds(h * (CHS * K) + 16 * g, 16)]
            for j in range(16):
                row = iv[j]
                s = 4 * g + (j // 4)
                for k in range(D // 16):
                    rows_b[s, pl.ds((j % 4) * D + k * 16, 16)] = (
                        table_v[row, pl.ds(k * 16, 16)])

        pltpu.async_copy(rows_b, out_sl, sem_b)

    def round_fn(t2, _):
        for b in range(2):
            do_chunk(t2 * 2 + b, *bufs[b])
        return 0

    lax.fori_loop(0, TCH // 2, round_fn, 0)

    for b in range(2):
        t = TCH - 2 + b
        bb = base_b + t // NCH
        out_sl = out_hbm.at[bb, pl.ds((t % NCH) * CHS, CHS), :]
        pltpu.make_async_copy(bufs[b][0], out_sl, bufs[b][1]).wait()


def kernel(byte_indices, W):
    idx_flat = byte_indices.reshape(-1).astype(jnp.int32)
    return _gather_kernel(idx_flat, W.astype(jnp.float32))


# speedup vs baseline: 7.5175x; 1.4233x over previous
import functools

import jax
import jax.numpy as jnp
from jax import lax
from jax.experimental import pallas as pl
from jax.experimental.pallas import tpu as pltpu
from jax.experimental.pallas import tpu_sc as plsc

NC = 2
NS = 16
NW = NC * NS

V = 256
D = 64
BATCH = 1024
SEQ = 200
K = 4
NB = BATCH // NW
RPB = SEQ * K
S0A, S0B = 0, 104
LNA, LNB = 104, 96

_mesh = plsc.VectorSubcoreMesh(core_axis_name="c", subcore_axis_name="s")


@functools.partial(
    pl.kernel,
    mesh=_mesh,
    out_type=jax.ShapeDtypeStruct((BATCH, SEQ, K * D), jnp.float32),
    scratch_types=[
        pltpu.VMEM((V, D), jnp.float32),
        pltpu.VMEM((RPB,), jnp.int32),
        pltpu.VMEM((LNA, K * D), jnp.float32),
        pltpu.VMEM((LNB, K * D), jnp.float32),
        pltpu.SemaphoreType.DMA,
        pltpu.SemaphoreType.DMA,
    ],
)
def _gather_kernel(idx_hbm, w_hbm, out_hbm, table_v, idx_b, rowsA, rowsB,
                   semA, semB):
    wid = lax.axis_index("s") * NC + lax.axis_index("c")
    base_b = wid * NB

    pltpu.sync_copy(w_hbm, table_v)

    slabs = ((rowsA, semA, S0A, LNA), (rowsB, semB, S0B, LNB))

    def do_batch(i, _):
        bb = base_b + i
        pltpu.sync_copy(idx_hbm.at[pl.ds(bb * RPB, RPB)], idx_b)

        for rows_b, sem_b, s0, ln in slabs:
            out_sl = out_hbm.at[bb, pl.ds(s0, ln), :]

            @pl.when(i >= 1)
            def _():
                pltpu.make_async_copy(rows_b, out_sl, sem_b).wait()

            @plsc.parallel_loop(0, ln * K // 16, unroll=2)
            def grp_fn(g):
                iv = idx_b[pl.ds(s0 * K + 16 * g, 16)]
                for j in range(16):
                    row = iv[j]
                    s = 4 * g + (j // 4)
                    for k in range(D // 16):
                        rows_b[s, pl.ds((j % 4) * D + k * 16, 16)] = (
                            table_v[row, pl.ds(k * 16, 16)])

            pltpu.async_copy(rows_b, out_sl, sem_b)
        return 0

    lax.fori_loop(0, NB, do_batch, 0)

    bb = base_b + NB - 1
    for rows_b, sem_b, s0, ln in slabs:
        out_sl = out_hbm.at[bb, pl.ds(s0, ln), :]
        pltpu.make_async_copy(rows_b, out_sl, sem_b).wait()


def kernel(byte_indices, W):
    idx_flat = byte_indices.reshape(-1).astype(jnp.int32)
    return _gather_kernel(idx_flat, W.astype(jnp.float32))


# speedup vs baseline: 7.6840x; 1.0221x over previous
import functools

import jax
import jax.numpy as jnp
from jax import lax
from jax.experimental import pallas as pl
from jax.experimental.pallas import tpu as pltpu
from jax.experimental.pallas import tpu_sc as plsc

NC = 2
NS = 16
NW = NC * NS

V = 256
D = 64
BATCH = 1024
SEQ = 200
K = 4
NB = BATCH // NW
RPB = SEQ * K
S0A, S0B = 0, 104
LNA, LNB = 104, 96

_mesh = plsc.VectorSubcoreMesh(core_axis_name="c", subcore_axis_name="s")


@functools.partial(
    pl.kernel,
    mesh=_mesh,
    out_type=jax.ShapeDtypeStruct((BATCH, SEQ, K * D), jnp.float32),
    scratch_types=[
        pltpu.VMEM((V, D), jnp.float32),
        pltpu.VMEM((RPB,), jnp.int32),
        pltpu.VMEM((RPB,), jnp.int32),
        pltpu.VMEM((LNA, K * D), jnp.float32),
        pltpu.VMEM((LNB, K * D), jnp.float32),
        pltpu.SemaphoreType.DMA,
        pltpu.SemaphoreType.DMA,
        pltpu.SemaphoreType.DMA,
    ],
)
def _gather_kernel(idx_hbm, w_hbm, out_hbm, table_v, idx0, idx1, rowsA, rowsB,
                   semA, semB, semI):
    wid = lax.axis_index("s") * NC + lax.axis_index("c")
    base_b = wid * NB

    pltpu.sync_copy(w_hbm, table_v)

    slabs = ((rowsA, semA, S0A, LNA), (rowsB, semB, S0B, LNB))
    ibufs = (idx0, idx1)

    pltpu.async_copy(idx_hbm.at[pl.ds(base_b * RPB, RPB)], idx0, semI)

    def do_batch2(i2, _):
        for p in range(2):
            do_batch(i2 * 2 + p, ibufs[p], ibufs[1 - p])
        return 0

    def do_batch(i, idx_b, idx_n):
        bb = base_b + i

        pltpu.make_async_copy(
            idx_hbm.at[pl.ds(bb * RPB, RPB)], idx_b, semI).wait()

        @pl.when(i < NB - 1)
        def _():
            pltpu.async_copy(
                idx_hbm.at[pl.ds((bb + 1) * RPB, RPB)], idx_n, semI)

        for rows_b, sem_b, s0, ln in slabs:
            out_sl = out_hbm.at[bb, pl.ds(s0, ln), :]

            @pl.when(i >= 1)
            def _():
                pltpu.make_async_copy(rows_b, out_sl, sem_b).wait()

            @plsc.parallel_loop(0, ln * K // 16, unroll=2)
            def grp_fn(g):
                iv = idx_b[pl.ds(s0 * K + 16 * g, 16)]
                for j in range(16):
                    row = iv[j]
                    s = 4 * g + (j // 4)
                    for k in range(D // 16):
                        rows_b[s, pl.ds((j % 4) * D + k * 16, 16)] = (
                            table_v[row, pl.ds(k * 16, 16)])

            pltpu.async_copy(rows_b, out_sl, sem_b)

    lax.fori_loop(0, NB // 2, do_batch2, 0)

    bb = base_b + NB - 1
    for rows_b, sem_b, s0, ln in slabs:
        out_sl = out_hbm.at[bb, pl.ds(s0, ln), :]
        pltpu.make_async_copy(rows_b, out_sl, sem_b).wait()


def kernel(byte_indices, W):
    idx_flat = byte_indices.reshape(-1).astype(jnp.int32)
    return _gather_kernel(idx_flat, W.astype(jnp.float32))


# speedup vs baseline: 15.3468x; 1.9973x over previous
import functools

import jax
import jax.numpy as jnp
from jax import lax
from jax.experimental import pallas as pl
from jax.experimental.pallas import tpu as pltpu
from jax.experimental.pallas import tpu_sc as plsc

NC = 2
NS = 16
NW = NC * NS

V = 256
D = 64
BATCH = 1024
SEQ = 200
K = 4
Q = 8
NPB = BATCH // Q
M = Q * K
SB = 8
NT = SEQ // SB
BW = 16

_mesh = plsc.VectorSubcoreMesh(core_axis_name="c", subcore_axis_name="s")


@functools.partial(
    pl.kernel,
    mesh=_mesh,
    out_type=jax.ShapeDtypeStruct((BATCH, SEQ, K * D), jnp.float32),
    scratch_types=[
        pltpu.VMEM((V, D), jnp.float32),
        pltpu.VMEM((SB, K, NPB), jnp.int32),
        pltpu.VMEM((SB, K, NPB), jnp.int32),
        pltpu.VMEM((BW, SB, K * D), jnp.float32),
        pltpu.VMEM((BW, SB, K * D), jnp.float32),
        pltpu.SemaphoreType.DMA,
        pltpu.SemaphoreType.DMA,
        pltpu.SemaphoreType.DMA,
    ],
)
def _gather_kernel(idx_hbm, w_hbm, out_hbm, table_v, idx0, idx1,
                   rows0, rows1, sem0, sem1, semI):
    wid = lax.axis_index("s") * NC + lax.axis_index("c")
    q = wid // 4
    n0 = (wid % 4) * 32

    pltpu.sync_copy(w_hbm, table_v)

    ibufs = (idx0, idx1)
    slabs = ((rows0, sem0), (rows1, sem1))

    def idx_src(t):
        return idx_hbm.at[pl.ds(SB * t, SB), pl.ds(K * q, K), :]

    pltpu.async_copy(idx_src(0), idx0, semI)

    def do_t(t, idx_b, idx_n):
        pltpu.make_async_copy(idx_src(t), idx_b, semI).wait()

        @pl.when(t < NT - 1)
        def _():
            pltpu.async_copy(idx_src(t + 1), idx_n, semI)

        for h in range(2):
            rows_b, sem_b = slabs[h]
            b0 = NPB * q + n0 + BW * h
            out_sl = out_hbm.at[pl.ds(b0, BW), pl.ds(SB * t, SB), :]

            @pl.when(t >= 1)
            def _():
                pltpu.make_async_copy(rows_b, out_sl, sem_b).wait()

            @plsc.parallel_loop(0, SB * K, unroll=2)
            def pair_fn(g):
                sl = g // K
                k = g % K
                iv = idx_b[sl, k, pl.ds(n0 + BW * h, BW)]
                for j in range(BW):
                    row = iv[j]
                    for kk in range(D // 16):
                        rows_b[j, sl, pl.ds(k * D + kk * 16, 16)] = (
                            table_v[row, pl.ds(kk * 16, 16)])

            pltpu.async_copy(rows_b, out_sl, sem_b)

    def do_t2(t2, _):
        for p in range(2):
            do_t(t2 * 2 + p, ibufs[p], ibufs[1 - p])
        return 0

    lax.fori_loop(0, NT // 2, do_t2, 0)
    do_t(NT - 1, ibufs[(NT - 1) % 2], ibufs[NT % 2])

    for h in range(2):
        rows_b, sem_b = slabs[h]
        b0 = NPB * q + n0 + BW * h
        out_sl = out_hbm.at[pl.ds(b0, BW), pl.ds(SB * (NT - 1), SB), :]
        pltpu.make_async_copy(rows_b, out_sl, sem_b).wait()


def kernel(byte_indices, W):
    idx = byte_indices.astype(jnp.int32)
    idx3 = idx.reshape(Q, NPB, SEQ, K).transpose(2, 0, 3, 1).reshape(SEQ, M, NPB)
    return _gather_kernel(idx3, W.astype(jnp.float32))
